# speedup vs baseline: 2.4411x; 2.4411x over previous
import functools

import jax
import jax.numpy as jnp
from jax.experimental import pallas as pl


def _neighbor(H, W, K):
    ws = int(round(K ** 0.5))
    off = ws // 2
    kh, kw = jnp.meshgrid(jnp.arange(ws), jnp.arange(ws), indexing="ij")
    kh = kh.reshape(-1)
    kw = kw.reshape(-1)
    hj = jnp.clip(jnp.arange(H) - off, 0, H - ws)[:, None] + kh[None, :]
    wj = jnp.clip(jnp.arange(W) - off, 0, W - ws)[:, None] + kw[None, :]
    return hj, wj


def _combine_body(a_ref, g_ref, pi_ref, o_ref, *, nsp):
    eps = 1e-10
    a = a_ref[...]
    g = g_ref[...]
    pi = pi_ref[...]
    m = jnp.max(a, axis=1, keepdims=True)
    e = jnp.exp(a - m)
    acc = jnp.zeros_like(a)
    for s in range(nsp):
        gs = g[s][None]
        den = eps + jnp.sum(e * gs, axis=1, keepdims=True)
        acc = acc + (pi[s][None] / den) * gs
    o_ref[...] = e * acc


def _reweight_one(attn_b, sims_b, sinds_b):
    HD, H, W, K = attn_b.shape
    NSP = sinds_b.shape[-1]
    HW = H * W
    hj, wj = _neighbor(H, W, K)

    sp = jnp.transpose(sinds_b, (2, 0, 1))
    hh = hj.T[None, :, :, None]
    ww = wj.T[None, :, None, :]
    G = sims_b[sp[:, None, :, :], hh, ww]
    G = G.reshape(NSP, K, HW)
    pi = sims_b[sp, jnp.arange(H)[None, :, None], jnp.arange(W)[None, None, :]]
    pi = pi.reshape(NSP, 1, HW)

    aT = jnp.transpose(attn_b, (0, 3, 1, 2)).reshape(HD, K, HW)

    P = 512
    grid = (HW // P,)
    out = pl.pallas_call(
        functools.partial(_combine_body, nsp=NSP),
        grid=grid,
        in_specs=[
            pl.BlockSpec((HD, K, P), lambda i: (0, 0, i)),
            pl.BlockSpec((NSP, K, P), lambda i: (0, 0, i)),
            pl.BlockSpec((NSP, 1, P), lambda i: (0, 0, i)),
        ],
        out_specs=pl.BlockSpec((HD, K, P), lambda i: (0, 0, i)),
        out_shape=jax.ShapeDtypeStruct((HD, K, HW), jnp.float32),
    )(aT, G, pi)
    return jnp.transpose(out.reshape(HD, K, H, W), (0, 2, 3, 1))


def kernel(attn, sims, sinds):
    B = attn.shape[0]
    outs = [_reweight_one(attn[b], sims[b], sinds[b]) for b in range(B)]
    return jnp.stack(outs, axis=0)


# speedup vs baseline: 2.4521x; 1.0045x over previous
import functools

import jax
import jax.numpy as jnp
from jax import lax
from jax.experimental import pallas as pl
from jax.experimental.pallas import tpu as pltpu
from jax.experimental.pallas import tpu_sc as plsc

NC = 2
NS = 16
NWRK = NC * NS

TH = 14
TW = 16


def _tree_reduce(xs, op):
    xs = list(xs)
    while len(xs) > 1:
        nxt = [op(xs[i], xs[i + 1]) for i in range(0, len(xs) - 1, 2)]
        if len(xs) % 2:
            nxt.append(xs[-1])
        xs = nxt
    return xs[0]


@functools.lru_cache(maxsize=None)
def _build_sc(H, W, S, HD, K, NSP):
    assert H % TH == 0 and W % TW == 0
    GH, GW = H // TH, W // TW
    NT = GH * GW
    assert NT % NWRK == 0
    TPW = NT // NWRK
    ws = int(round(K ** 0.5))
    WNH, WNW = TH + ws - 1, TW + ws - 1
    ROWW = WNW * S
    WROW = W * S
    RK = TW * K
    SR = TW * NSP
    AROW = W * K
    NROW = W * NSP
    eps = 1e-10

    mesh = plsc.VectorSubcoreMesh(
        core_axis_name="c", subcore_axis_name="s",
        num_cores=NC, num_subcores=NS)

    @functools.partial(
        pl.kernel,
        out_type=jax.ShapeDtypeStruct((HD * H * W * K,), jnp.float32),
        mesh=mesh,
        compiler_params=pltpu.CompilerParams(needs_layout_passes=False),
        scratch_types=[
            pltpu.VMEM((WNH * ROWW,), jnp.float32),
            pltpu.VMEM((TH * SR,), jnp.int32),
            pltpu.VMEM((NSP * K * 16,), jnp.float32),
            pltpu.VMEM((NSP * 16,), jnp.float32),
            pltpu.VMEM((HD * RK,), jnp.float32),
            pltpu.VMEM((HD * RK,), jnp.float32),
            pltpu.VMEM((HD * RK,), jnp.float32),
            pltpu.VMEM((HD * RK,), jnp.float32),
            pltpu.SemaphoreType.DMA,
            pltpu.SemaphoreType.DMA,
            pltpu.SemaphoreType.DMA,
            pltpu.SemaphoreType.DMA,
        ],
    )
    def sc_reweight(simsF, attnF, sindsF, outF,
                    win, sindb, gb, pib, abuf0, abuf1, obuf0, obuf1,
                    sem_in, sem_a, sem_o0, sem_o1):
        cid = lax.axis_index("c")
        sid = lax.axis_index("s")
        wid = sid * NC + cid
        iota = lax.iota(jnp.int32, 16)
        ia = iota * K

        def run_chunk(it, j, par, r0, c0, rs2, cs2, fcol, pcol,
                      abuf, obuf, sem_o):
            i = 2 * j + par
            h = r0 + i
            a_handles = []
            for hd in range(HD):
                aoff = (hd * H + h) * AROW + c0 * K
                a_handles.append(pltpu.async_copy(
                    attnF.at[pl.ds(aoff, RK)],
                    abuf.at[pl.ds(hd * RK, RK)], sem_a))

            h0 = jnp.clip(h - 2, 0, H - ws)
            lr = h0 - rs2
            lrp = h - rs2
            sbase = i * SR + iota * NSP
            svs = [plsc.load_gather(sindb, [sbase + s]) for s in range(NSP)]
            for s in range(NSP):
                vb = fcol + svs[s]
                for k in range(K):
                    kh, kw = k // ws, k % ws
                    g = plsc.load_gather(
                        win, [vb + ((lr + kh) * ROWW + kw * S)])
                    gb[pl.ds((s * K + k) * 16, 16)] = g
                pi_s = plsc.load_gather(win, [pcol + svs[s] + lrp * ROWW])
                pib[pl.ds(s * 16, 16)] = pi_s

            @pl.when((it > 0) | (j > 0))
            def _drain_out():
                for hd in range(HD):
                    pltpu.make_async_copy(
                        obuf.at[pl.ds(hd * RK, RK)],
                        outF.at[pl.ds(hd * RK, RK)], sem_o).wait()

            for hcp in a_handles:
                hcp.wait()

            def hd_body(hd, carry2):
                abase = hd * RK
                aks = [plsc.load_gather(abuf, [abase + ia + k])
                       for k in range(K)]
                m = _tree_reduce(aks, jnp.maximum)
                es = [jnp.exp(a - m) for a in aks]
                acc = [jnp.zeros((16,), jnp.float32)] * K
                for s in range(NSP):
                    gks = [gb[pl.ds((s * K + k) * 16, 16)] for k in range(K)]
                    den = _tree_reduce([es[k] * gks[k] for k in range(K)],
                                       lambda x, y: x + y) + eps
                    r = pib[pl.ds(s * 16, 16)] / den
                    acc = [acc[k] + r * gks[k] for k in range(K)]
                for k in range(K):
                    plsc.store_scatter(obuf, [abase + ia + k],
                                       es[k] * acc[k])
                return carry2
            lax.fori_loop(0, HD, hd_body, 0)

            for hd in range(HD):
                ooff = (hd * H + h) * AROW + c0 * K
                pltpu.async_copy(obuf.at[pl.ds(hd * RK, RK)],
                                 outF.at[pl.ds(ooff, RK)], sem_o)

        def tile_body(it, carry):
            t = wid * TPW + it
            tr = t % GH
            tc = t // GH
            r0 = tr * TH
            c0 = tc * TW
            rs2 = jnp.clip(r0 - 2, 0, H - WNH)
            cs2 = jnp.clip(c0 - 2, 0, W - WNW)

            in_handles = []
            for r in range(WNH):
                in_handles.append(pltpu.async_copy(
                    simsF.at[pl.ds((rs2 + r) * WROW + cs2 * S, ROWW)],
                    win.at[pl.ds(r * ROWW, ROWW)], sem_in))
            for i in range(TH):
                in_handles.append(pltpu.async_copy(
                    sindsF.at[pl.ds((r0 + i) * NROW + c0 * NSP, SR)],
                    sindb.at[pl.ds(i * SR, SR)], sem_in))
            for hcp in in_handles:
                hcp.wait()

            w = c0 + iota
            w0 = jnp.clip(w - 2, 0, W - ws)
            fcol = (w0 - cs2) * S
            pcol = (w - cs2) * S

            def pair_body(j, carry2):
                run_chunk(it, j, 0, r0, c0, rs2, cs2, fcol, pcol,
                          abuf0, obuf0, sem_o0)
                run_chunk(it, j, 1, r0, c0, rs2, cs2, fcol, pcol,
                          abuf1, obuf1, sem_o1)
                return carry2
            lax.fori_loop(0, TH // 2, pair_body, 0)
            return carry

        lax.fori_loop(0, TPW, tile_body, 0)

        for obuf, sem_o in ((obuf0, sem_o0), (obuf1, sem_o1)):
            for hd in range(HD):
                pltpu.make_async_copy(
                    obuf.at[pl.ds(hd * RK, RK)],
                    outF.at[pl.ds(hd * RK, RK)], sem_o).wait()

    return sc_reweight


def kernel(attn, sims, sinds):
    B, HD, H, W, K = attn.shape
    S = sims.shape[1]
    NSP = sinds.shape[-1]
    fn = _build_sc(H, W, S, HD, K, NSP)
    outs = []
    for b in range(B):
        simsF = jnp.transpose(sims[b], (1, 2, 0)).reshape(H * W * S)
        attnF = attn[b].reshape(HD * H * W * K)
        sindsF = sinds[b].reshape(H * W * NSP)
        outF = fn(simsF, attnF, sindsF)
        outs.append(outF.reshape(HD, H, W, K))
    return jnp.stack(outs, axis=0)


# speedup vs baseline: 2.8813x; 1.1751x over previous
import functools

import jax
import jax.numpy as jnp
from jax import lax
from jax.experimental import pallas as pl
from jax.experimental.pallas import tpu as pltpu
from jax.experimental.pallas import tpu_sc as plsc

NC = 2
NS = 16
NWRK = NC * NS

TH = 14
TW = 16


def _tree_reduce(xs, op):
    xs = list(xs)
    while len(xs) > 1:
        nxt = [op(xs[i], xs[i + 1]) for i in range(0, len(xs) - 1, 2)]
        if len(xs) % 2:
            nxt.append(xs[-1])
        xs = nxt
    return xs[0]


@functools.lru_cache(maxsize=None)
def _build_sc(H, W, S, HD, K, NSP):
    assert H % TH == 0 and W % TW == 0
    GH, GW = H // TH, W // TW
    NT = GH * GW
    assert NT % NWRK == 0
    TPW = NT // NWRK
    ws = int(round(K ** 0.5))
    WNH, WNW = TH + ws - 1, TW + ws - 1
    ROWW = WNW * S
    WROW = W * S
    RK = TW * K
    SR = TW * NSP
    AROW = W * K
    NROW = W * NSP
    eps = 1e-10

    mesh = plsc.VectorSubcoreMesh(
        core_axis_name="c", subcore_axis_name="s",
        num_cores=NC, num_subcores=NS)

    @functools.partial(
        pl.kernel,
        out_type=jax.ShapeDtypeStruct((HD * H * W * K,), jnp.float32),
        mesh=mesh,
        compiler_params=pltpu.CompilerParams(needs_layout_passes=False),
        scratch_types=[
            pltpu.VMEM((WNH * ROWW,), jnp.float32),
            pltpu.VMEM((TH * SR,), jnp.int32),
            pltpu.VMEM((NSP * K * 16,), jnp.float32),
            pltpu.VMEM((NSP * 16,), jnp.int32),
            pltpu.VMEM((NSP * 16,), jnp.int32),
            pltpu.VMEM((HD * NSP * 16,), jnp.float32),
            pltpu.VMEM((HD * K * 16,), jnp.float32),
            pltpu.VMEM((HD * RK,), jnp.float32),
            pltpu.VMEM((HD * RK,), jnp.float32),
            pltpu.VMEM((HD * RK,), jnp.float32),
            pltpu.VMEM((HD * RK,), jnp.float32),
            pltpu.SemaphoreType.DMA,
            pltpu.SemaphoreType.DMA,
            pltpu.SemaphoreType.DMA,
            pltpu.SemaphoreType.DMA,
        ],
    )
    def sc_reweight(simsF, attnF, sindsF, outF,
                    win, sindb, gb, vbb, pvb, rsb, eb,
                    abuf0, abuf1, obuf0, obuf1,
                    sem_in, sem_a, sem_o0, sem_o1):
        cid = lax.axis_index("c")
        sid = lax.axis_index("s")
        wid = sid * NC + cid
        iota = lax.iota(jnp.int32, 16)
        ia = iota * K

        def run_chunk(it, j, par, r0, c0, rs2, cs2, fcol, pcol,
                      abuf, obuf, sem_o):
            i = 2 * j + par
            h = r0 + i
            a_handles = []
            for hd in range(HD):
                aoff = (hd * H + h) * AROW + c0 * K
                a_handles.append(pltpu.async_copy(
                    attnF.at[pl.ds(aoff, RK)],
                    abuf.at[pl.ds(hd * RK, RK)], sem_a))

            h0 = jnp.clip(h - 2, 0, H - ws)
            lr = h0 - rs2
            lrp = h - rs2
            sbase = i * SR + iota * NSP
            for s in range(NSP):
                sv = plsc.load_gather(sindb, [sbase + s])
                vbb[pl.ds(s * 16, 16)] = fcol + sv
                pvb[pl.ds(s * 16, 16)] = pcol + sv + lrp * ROWW

            @pl.when((it > 0) | (j > 0))
            def _drain_out():
                for hd in range(HD):
                    pltpu.make_async_copy(
                        obuf.at[pl.ds(hd * RK, RK)],
                        outF.at[pl.ds(hd * RK, RK)], sem_o).wait()

            for hcp in a_handles:
                hcp.wait()

            @plsc.parallel_loop(0, K, unroll=2)
            def _eprep(k):
                aks = [plsc.load_gather(abuf, [hd * RK + ia + k])
                       for hd in range(HD)]
                for hd in range(HD):
                    eb[pl.ds((hd * K + k) * 16, 16)] = jnp.exp(aks[hd])

            def _den(s, carry2):
                vb = vbb[pl.ds(s * 16, 16)]
                dens = [None] * HD
                for kh in range(ws):
                    offr = (lr + kh) * ROWW
                    gs = [plsc.load_gather(win, [vb + (offr + kw * S)])
                          for kw in range(ws)]
                    for kw in range(ws):
                        gb[pl.ds(s * (K * 16) + (kh * ws + kw) * 16, 16)] = (
                            gs[kw])
                    for hd in range(HD):
                        es = [eb[pl.ds((hd * K + kh * ws + kw) * 16, 16)]
                              for kw in range(ws)]
                        p = _tree_reduce(
                            [es[kw] * gs[kw] for kw in range(ws)],
                            lambda x, y: x + y)
                        dens[hd] = p if kh == 0 else dens[hd] + p
                piv = plsc.load_gather(win, [pvb[pl.ds(s * 16, 16)]])
                for hd in range(HD):
                    rsb[pl.ds((hd * NSP + s) * 16, 16)] = (
                        piv / (dens[hd] + eps))
                return carry2
            lax.fori_loop(0, NSP, _den, 0)

            for pair in ((0, 1), (2, 3)):
                rs = {hd: [rsb[pl.ds((hd * NSP + s) * 16, 16)]
                           for s in range(NSP)] for hd in pair}

                def pass2(k, rs=rs, pair=pair):
                    gs = [gb[pl.ds(s * (K * 16) + k * 16, 16)]
                          for s in range(NSP)]
                    for hd in pair:
                        acc = _tree_reduce(
                            [rs[hd][s] * gs[s] for s in range(NSP)],
                            lambda x, y: x + y)
                        out_k = eb[pl.ds((hd * K + k) * 16, 16)] * acc
                        plsc.store_scatter(obuf, [hd * RK + ia + k], out_k)
                plsc.parallel_loop(0, K, unroll=2)(pass2)

            for hd in range(HD):
                ooff = (hd * H + h) * AROW + c0 * K
                pltpu.async_copy(obuf.at[pl.ds(hd * RK, RK)],
                                 outF.at[pl.ds(ooff, RK)], sem_o)

        def tile_body(it, carry):
            t = wid * TPW + it
            tr = t % GH
            tc = t // GH
            r0 = tr * TH
            c0 = tc * TW
            rs2 = jnp.clip(r0 - 2, 0, H - WNH)
            cs2 = jnp.clip(c0 - 2, 0, W - WNW)

            in_handles = []
            for r in range(WNH):
                in_handles.append(pltpu.async_copy(
                    simsF.at[pl.ds((rs2 + r) * WROW + cs2 * S, ROWW)],
                    win.at[pl.ds(r * ROWW, ROWW)], sem_in))
            for i in range(TH):
                in_handles.append(pltpu.async_copy(
                    sindsF.at[pl.ds((r0 + i) * NROW + c0 * NSP, SR)],
                    sindb.at[pl.ds(i * SR, SR)], sem_in))
            for hcp in in_handles:
                hcp.wait()

            w = c0 + iota
            w0 = jnp.clip(w - 2, 0, W - ws)
            fcol = (w0 - cs2) * S
            pcol = (w - cs2) * S

            def pair_body(j, carry2):
                run_chunk(it, j, 0, r0, c0, rs2, cs2, fcol, pcol,
                          abuf0, obuf0, sem_o0)
                run_chunk(it, j, 1, r0, c0, rs2, cs2, fcol, pcol,
                          abuf1, obuf1, sem_o1)
                return carry2
            lax.fori_loop(0, TH // 2, pair_body, 0)
            return carry

        lax.fori_loop(0, TPW, tile_body, 0)

        for obuf, sem_o in ((obuf0, sem_o0), (obuf1, sem_o1)):
            for hd in range(HD):
                pltpu.make_async_copy(
                    obuf.at[pl.ds(hd * RK, RK)],
                    outF.at[pl.ds(hd * RK, RK)], sem_o).wait()

    return sc_reweight


def kernel(attn, sims, sinds):
    B, HD, H, W, K = attn.shape
    S = sims.shape[1]
    NSP = sinds.shape[-1]
    fn = _build_sc(H, W, S, HD, K, NSP)
    outs = []
    for b in range(B):
        simsF = jnp.transpose(sims[b], (1, 2, 0)).reshape(H * W * S)
        attnF = attn[b].reshape(HD * H * W * K)
        sindsF = sinds[b].reshape(H * W * NSP)
        outF = fn(simsF, attnF, sindsF)
        outs.append(outF.reshape(HD, H, W, K))
    return jnp.stack(outs, axis=0)


# speedup vs baseline: 3.0918x; 1.0730x over previous
import functools

import jax
import jax.numpy as jnp
from jax import lax
from jax.experimental import pallas as pl
from jax.experimental.pallas import tpu as pltpu
from jax.experimental.pallas import tpu_sc as plsc

NC = 2
NS = 16
NWRK = NC * NS

TH = 14
TW = 16


def _tree_reduce(xs, op):
    xs = list(xs)
    while len(xs) > 1:
        nxt = [op(xs[i], xs[i + 1]) for i in range(0, len(xs) - 1, 2)]
        if len(xs) % 2:
            nxt.append(xs[-1])
        xs = nxt
    return xs[0]


@functools.lru_cache(maxsize=None)
def _build_sc(H, W, S, HD, K, NSP):
    assert H % TH == 0 and W % TW == 0
    GH, GW = H // TH, W // TW
    NT = GH * GW
    assert NT % NWRK == 0
    TPW = NT // NWRK
    ws = int(round(K ** 0.5))
    WNH, WNW = TH + ws - 1, TW + ws - 1
    ROWW = WNW * S
    WROW = W * S
    RK = TW * K
    SR = TW * NSP
    AROW = W * K
    NROW = W * NSP
    eps = 1e-10

    mesh = plsc.VectorSubcoreMesh(
        core_axis_name="c", subcore_axis_name="s",
        num_cores=NC, num_subcores=NS)

    @functools.partial(
        pl.kernel,
        out_type=jax.ShapeDtypeStruct((HD * H * W * K,), jnp.float32),
        mesh=mesh,
        compiler_params=pltpu.CompilerParams(needs_layout_passes=False),
        scratch_types=[
            pltpu.VMEM((WNH * ROWW,), jnp.float32),
            pltpu.VMEM((TH * SR,), jnp.int32),
            pltpu.VMEM((NSP * K * 16,), jnp.float32),
            pltpu.VMEM((NSP * 16,), jnp.int32),
            pltpu.VMEM((NSP * 16,), jnp.int32),
            pltpu.VMEM((HD * NSP * 16,), jnp.float32),
            pltpu.VMEM((HD * K * 16,), jnp.float32),
            pltpu.VMEM((HD * RK,), jnp.float32),
            pltpu.VMEM((HD * RK,), jnp.float32),
            pltpu.VMEM((HD * RK,), jnp.float32),
            pltpu.VMEM((HD * RK,), jnp.float32),
            pltpu.SemaphoreType.DMA,
            pltpu.SemaphoreType.DMA,
            pltpu.SemaphoreType.DMA,
            pltpu.SemaphoreType.DMA,
            pltpu.SemaphoreType.DMA,
            pltpu.SemaphoreType.DMA,
        ],
    )
    def sc_reweight(simsF, attnF, sindsF, outF,
                    win, sindb, gb, vbb, pvb, rsb, eb,
                    abuf0, abuf1, obuf0, obuf1,
                    sem_in, sem_in2, sem_a0, sem_a1, sem_o0, sem_o1):
        cid = lax.axis_index("c")
        sid = lax.axis_index("s")
        wid = sid * NC + cid
        iota = lax.iota(jnp.int32, 16)
        ia = iota * K

        def run_chunk(it, j, par, r0, c0, rs2, cs2, fcol, pcol,
                      abuf, obuf, sem_a, sem_o):
            i = 2 * j + par
            h = r0 + i

            h0 = jnp.clip(h - 2, 0, H - ws)
            lr = h0 - rs2
            lrp = h - rs2
            sbase = i * SR + iota * NSP
            for s in range(NSP):
                sv = plsc.load_gather(sindb, [sbase + s])
                vbb[pl.ds(s * 16, 16)] = fcol + sv
                pvb[pl.ds(s * 16, 16)] = pcol + sv + lrp * ROWW

            @pl.when((it > 0) | (j > 0))
            def _drain_out():
                for hd in range(HD):
                    pltpu.make_async_copy(
                        obuf.at[pl.ds(hd * RK, RK)],
                        outF.at[pl.ds(hd * RK, RK)], sem_o).wait()

            for hd in range(HD):
                pltpu.make_async_copy(
                    attnF.at[pl.ds((hd * H + h) * AROW + c0 * K, RK)],
                    abuf.at[pl.ds(hd * RK, RK)], sem_a).wait()

            @plsc.parallel_loop(0, K, unroll=2)
            def _eprep(k):
                aks = [plsc.load_gather(abuf, [hd * RK + ia + k])
                       for hd in range(HD)]
                for hd in range(HD):
                    eb[pl.ds((hd * K + k) * 16, 16)] = jnp.exp(aks[hd])

            @pl.when(j < TH // 2 - 1)
            def _prefetch_attn():
                for hd in range(HD):
                    aoff = (hd * H + h + 2) * AROW + c0 * K
                    pltpu.async_copy(attnF.at[pl.ds(aoff, RK)],
                                     abuf.at[pl.ds(hd * RK, RK)], sem_a)

            def _den(s, carry2):
                vb = vbb[pl.ds(s * 16, 16)]
                dens = [None] * HD
                for kh in range(ws):
                    offr = (lr + kh) * ROWW
                    gs = [plsc.load_gather(win, [vb + (offr + kw * S)])
                          for kw in range(ws)]
                    for kw in range(ws):
                        gb[pl.ds(s * (K * 16) + (kh * ws + kw) * 16, 16)] = (
                            gs[kw])
                    for hd in range(HD):
                        es = [eb[pl.ds((hd * K + kh * ws + kw) * 16, 16)]
                              for kw in range(ws)]
                        p = _tree_reduce(
                            [es[kw] * gs[kw] for kw in range(ws)],
                            lambda x, y: x + y)
                        dens[hd] = p if kh == 0 else dens[hd] + p
                piv = plsc.load_gather(win, [pvb[pl.ds(s * 16, 16)]])
                for hd in range(HD):
                    rsb[pl.ds((hd * NSP + s) * 16, 16)] = (
                        piv / (dens[hd] + eps))
                return carry2
            lax.fori_loop(0, NSP, _den, 0)

            for pair in ((0, 1), (2, 3)):
                rs = {hd: [rsb[pl.ds((hd * NSP + s) * 16, 16)]
                           for s in range(NSP)] for hd in pair}

                def pass2(k, rs=rs, pair=pair):
                    gs = [gb[pl.ds(s * (K * 16) + k * 16, 16)]
                          for s in range(NSP)]
                    for hd in pair:
                        acc = _tree_reduce(
                            [rs[hd][s] * gs[s] for s in range(NSP)],
                            lambda x, y: x + y)
                        out_k = eb[pl.ds((hd * K + k) * 16, 16)] * acc
                        plsc.store_scatter(obuf, [hd * RK + ia + k], out_k)
                plsc.parallel_loop(0, K, unroll=2)(pass2)

            for hd in range(HD):
                ooff = (hd * H + h) * AROW + c0 * K
                pltpu.async_copy(obuf.at[pl.ds(hd * RK, RK)],
                                 outF.at[pl.ds(ooff, RK)], sem_o)

        def tile_body(it, carry):
            t = wid * TPW + it
            tr = t % GH
            tc = t // GH
            r0 = tr * TH
            c0 = tc * TW
            rs2 = jnp.clip(r0 - 2, 0, H - WNH)
            cs2 = jnp.clip(c0 - 2, 0, W - WNW)

            in_handles = []
            for r in range(WNH):
                in_handles.append(pltpu.async_copy(
                    simsF.at[pl.ds((rs2 + r) * WROW + cs2 * S, ROWW)],
                    win.at[pl.ds(r * ROWW, ROWW)],
                    sem_in if r < 8 else sem_in2))
            for i in range(TH):
                in_handles.append(pltpu.async_copy(
                    sindsF.at[pl.ds((r0 + i) * NROW + c0 * NSP, SR)],
                    sindb.at[pl.ds(i * SR, SR)], sem_in))
            for par, abuf, sem_a in ((0, abuf0, sem_a0), (1, abuf1, sem_a1)):
                for hd in range(HD):
                    aoff = (hd * H + r0 + par) * AROW + c0 * K
                    pltpu.async_copy(attnF.at[pl.ds(aoff, RK)],
                                     abuf.at[pl.ds(hd * RK, RK)], sem_a)
            for hcp in in_handles[:8] + in_handles[WNH:]:
                hcp.wait()

            w = c0 + iota
            w0 = jnp.clip(w - 2, 0, W - ws)
            fcol = (w0 - cs2) * S
            pcol = (w - cs2) * S

            def pair_body(j, carry2):
                @pl.when(j == 1)
                def _drain_win_tail():
                    for r in range(8, WNH):
                        pltpu.make_async_copy(
                            simsF.at[pl.ds(cs2 * S, ROWW)],
                            win.at[pl.ds(r * ROWW, ROWW)], sem_in2).wait()
                run_chunk(it, j, 0, r0, c0, rs2, cs2, fcol, pcol,
                          abuf0, obuf0, sem_a0, sem_o0)
                run_chunk(it, j, 1, r0, c0, rs2, cs2, fcol, pcol,
                          abuf1, obuf1, sem_a1, sem_o1)
                return carry2
            lax.fori_loop(0, TH // 2, pair_body, 0)
            return carry

        lax.fori_loop(0, TPW, tile_body, 0)

        for obuf, sem_o in ((obuf0, sem_o0), (obuf1, sem_o1)):
            for hd in range(HD):
                pltpu.make_async_copy(
                    obuf.at[pl.ds(hd * RK, RK)],
                    outF.at[pl.ds(hd * RK, RK)], sem_o).wait()

    return sc_reweight


def kernel(attn, sims, sinds):
    B, HD, H, W, K = attn.shape
    S = sims.shape[1]
    NSP = sinds.shape[-1]
    fn = _build_sc(H, W, S, HD, K, NSP)
    outs = []
    for b in range(B):
        simsF = jnp.transpose(sims[b], (1, 2, 0)).reshape(H * W * S)
        attnF = attn[b].reshape(HD * H * W * K)
        sindsF = sinds[b].reshape(H * W * NSP)
        outF = fn(simsF, attnF, sindsF)
        outs.append(outF.reshape(HD, H, W, K))
    return jnp.stack(outs, axis=0)
